# baseline (speedup 1.0000x reference)
import jax
import jax.numpy as jnp
from jax import lax
from jax.experimental import pallas as pl
from jax.experimental.pallas import tpu as pltpu
from jax.experimental.pallas import tpu_sc as plsc

_N = 1000000
_D = 16
_B = 16384

_NC = 2
_NS = 16
_NW = _NC * _NS
_BPW = _B // _NW
_RPW = 2 * _BPW
_CH = _RPW // 128
_G = _BPW // 16


def _sc_uu_body(w_hbm, idx_hbm, out_hbm, pv, wr_v, uu_v, sem):
    wid = lax.axis_index("s") * _NC + lax.axis_index("c")

    pltpu.sync_copy(idx_hbm.at[wid], pv)

    copies = []
    for c in range(_CH):
        copies.append(pltpu.async_copy(
            w_hbm.at[pv.at[c]], wr_v.at[pl.ds(c * 128, 128)], sem))
    for cp in copies:
        cp.wait()

    def group_body(g, carry):
        pairs = g * 16 + lax.iota(jnp.int32, 16)
        rows_i = 2 * pairs
        rows_j = rows_i + 1
        sii = jnp.zeros((16,), jnp.float32)
        sjj = jnp.zeros((16,), jnp.float32)
        sdd = jnp.zeros((16,), jnp.float32)
        for d in range(_D):
            cols = jnp.full((16,), d, jnp.int32)
            vi = plsc.load_gather(wr_v, [rows_i, cols])
            vj = plsc.load_gather(wr_v, [rows_j, cols])
            diff = vi - vj
            sii = sii + vi * vi
            sjj = sjj + vj * vj
            sdd = sdd + diff * diff
        z = 2.0 * sdd
        denom = (1.0 - sii) * (1.0 - sjj)
        uu = 1.0 + z / denom
        uu_v[pl.ds(g * 16, 16)] = uu
        return carry

    lax.fori_loop(0, _G, group_body, 0)

    pltpu.sync_copy(uu_v, out_hbm.at[pl.ds(wid * _BPW, _BPW)])


@jax.jit
def _sc_uu(w, idx3):
    mesh = plsc.VectorSubcoreMesh(core_axis_name="c", subcore_axis_name="s")
    return pl.kernel(
        _sc_uu_body,
        mesh=mesh,
        compiler_params=pltpu.CompilerParams(
            needs_layout_passes=False, use_tc_tiling_on_sc=False),
        out_type=jax.ShapeDtypeStruct((_B,), jnp.float32),
        scratch_types=[
            pltpu.VMEM((_CH, 128), jnp.int32),
            pltpu.VMEM((_RPW, _D), jnp.float32),
            pltpu.VMEM((_BPW,), jnp.float32),
            pltpu.SemaphoreType.DMA,
        ],
    )(w, idx3)


def _acosh_body(uu_ref, out_ref):
    uu = uu_ref[...]
    out_ref[...] = jnp.log(uu + jnp.sqrt(uu * uu - 1.0))


@jax.jit
def _tc_acosh(uu):
    return pl.pallas_call(
        _acosh_body,
        out_shape=jax.ShapeDtypeStruct(uu.shape, jnp.float32),
    )(uu)


def kernel(w, idx):
    idx3 = idx.astype(jnp.int32).reshape(_NW, _CH, 128)
    uu = _sc_uu(w, idx3)
    return _tc_acosh(uu)
